# baseline (speedup 1.0000x reference)
import functools

import jax
import jax.numpy as jnp
from jax import lax
from jax.experimental import pallas as pl
from jax.experimental.pallas import tpu as pltpu
from jax.experimental.pallas import tpu_sc as plsc

N = 10000
E = 320000
D = 128
NC = 2
NS = 16
NW = NC * NS
EPT = E // NW
K = 100
CH = EPT // K
NP = 10240
RPS = NP // NS


@functools.cache
def _get_sc_kernels():
    mesh = plsc.VectorSubcoreMesh(core_axis_name="c", subcore_axis_name="s")

    @functools.partial(
        pl.kernel,
        out_type=jax.ShapeDtypeStruct((NC, NP, D), jnp.float32),
        mesh=mesh,
        scratch_types=[
            pltpu.VMEM((CH, K), jnp.int32),
            pltpu.VMEM((K, D), jnp.float32),
            pltpu.VMEM_SHARED((NP, D), jnp.float32),
        ],
    )
    def _sc_degree(row_hbm, ones_hbm, zero_hbm, out_hbm, row_v, ones_v, acc):
        c = lax.axis_index("c")
        s = lax.axis_index("s")
        cid = c * NS + s
        pltpu.sync_copy(zero_hbm, acc.at[pl.ds(s * RPS, RPS)])
        pltpu.sync_copy(ones_hbm, ones_v)
        pltpu.sync_copy(row_hbm.at[cid], row_v)
        plsc.subcore_barrier()

        def body(j, _):
            pltpu.sync_copy(ones_v, acc.at[row_v.at[j]], add=True)
            return ()

        lax.fori_loop(0, CH, body, (), unroll=False)
        plsc.subcore_barrier()
        pltpu.sync_copy(acc.at[pl.ds(s * RPS, RPS)],
                        out_hbm.at[c, pl.ds(s * RPS, RPS)])

    @functools.partial(
        pl.kernel,
        out_type=jax.ShapeDtypeStruct((NC, NP, D), jnp.float32),
        mesh=mesh,
        scratch_types=[
            pltpu.VMEM((CH, K), jnp.int32),
            pltpu.VMEM((CH, K), jnp.int32),
            pltpu.VMEM((K, D), jnp.float32),
            pltpu.VMEM_SHARED((NP, D), jnp.float32),
            pltpu.SemaphoreType.DMA,
        ],
    )
    def _sc_messages(hsc_hbm, col_hbm, row_hbm, zero_hbm, out_hbm,
                     col_v, row_v, rows_v, acc, sem):
        c = lax.axis_index("c")
        s = lax.axis_index("s")
        cid = c * NS + s
        pltpu.sync_copy(zero_hbm, acc.at[pl.ds(s * RPS, RPS)])
        pltpu.sync_copy(col_hbm.at[cid], col_v)
        pltpu.sync_copy(row_hbm.at[cid], row_v)
        plsc.subcore_barrier()

        def body(j, _):
            pltpu.async_copy(hsc_hbm.at[col_v.at[j]], rows_v, sem).wait()
            pltpu.sync_copy(rows_v, acc.at[row_v.at[j]], add=True)
            return ()

        lax.fori_loop(0, CH, body, (), unroll=False)
        plsc.subcore_barrier()
        pltpu.sync_copy(acc.at[pl.ds(s * RPS, RPS)],
                        out_hbm.at[c, pl.ds(s * RPS, RPS)])

    return _sc_degree, _sc_messages


BLK = 2000
GRID = N // BLK


def _tc_input_body(x_ref, w_ref, b_ref, dp_ref, h_ref, hsc_ref, mean_ref, acc):
    i = pl.program_id(0)
    h = jnp.dot(x_ref[...], w_ref[...],
                preferred_element_type=jnp.float32) + b_ref[...]
    deg = dp_ref[0, :, 0:1] + dp_ref[1, :, 0:1] + 1.0
    dis = lax.rsqrt(deg)
    h_ref[...] = h
    hsc_ref[...] = h * dis

    @pl.when(i == 0)
    def _():
        acc[...] = jnp.zeros_like(acc)

    acc[...] += jnp.sum(h, axis=0, keepdims=True)

    @pl.when(i == GRID - 1)
    def _():
        mean_ref[...] = acc[...] * (1.0 / N)


def _tc_layer_body(h_ref, mp_ref, dp_ref, mean_ref, gw_ref, gb_ref,
                   w_ref, b_ref, gamma_ref, beta_ref,
                   hn_ref, hsc_ref, mean_out_ref, acc):
    i = pl.program_id(0)
    deg = dp_ref[0, :, 0:1] + dp_ref[1, :, 0:1] + 1.0
    dis = lax.rsqrt(deg)
    msgs = (mp_ref[0] + mp_ref[1]) * dis
    ctx = jnp.dot(mean_ref[...], gw_ref[...],
                  preferred_element_type=jnp.float32) + gb_ref[...]
    z = jnp.dot(h_ref[...] + msgs, w_ref[...],
                preferred_element_type=jnp.float32) + b_ref[...] + ctx
    z = jnp.maximum(z, 0.0)
    mu = jnp.mean(z, axis=-1, keepdims=True)
    var = jnp.mean(jnp.square(z - mu), axis=-1, keepdims=True)
    hn = (z - mu) * lax.rsqrt(var + 1e-5) * gamma_ref[...] + beta_ref[...]
    hn_ref[...] = hn
    hsc_ref[...] = hn * dis

    @pl.when(i == 0)
    def _():
        acc[...] = jnp.zeros_like(acc)

    acc[...] += jnp.sum(hn, axis=0, keepdims=True)

    @pl.when(i == GRID - 1)
    def _():
        mean_out_ref[...] = acc[...] * (1.0 / N)


_row_spec = pl.BlockSpec((BLK, D), lambda i: (i, 0))
_full_spec = pl.BlockSpec((D, D), lambda i: (0, 0))
_vec_spec = pl.BlockSpec((1, D), lambda i: (0, 0))
_dp_spec = pl.BlockSpec((NC, BLK, D), lambda i: (0, i, 0))
_mp_spec = pl.BlockSpec((NC, BLK, D), lambda i: (0, i, 0))

_tc_input = pl.pallas_call(
    _tc_input_body,
    grid=(GRID,),
    in_specs=[_row_spec, _full_spec, _vec_spec, _dp_spec],
    out_specs=[_row_spec, _row_spec, _vec_spec],
    out_shape=[
        jax.ShapeDtypeStruct((N, D), jnp.float32),
        jax.ShapeDtypeStruct((N, D), jnp.float32),
        jax.ShapeDtypeStruct((1, D), jnp.float32),
    ],
    scratch_shapes=[pltpu.VMEM((1, D), jnp.float32)],
)

_tc_layer = pl.pallas_call(
    _tc_layer_body,
    grid=(GRID,),
    in_specs=[_row_spec, _mp_spec, _dp_spec, _vec_spec, _full_spec, _vec_spec,
              _full_spec, _vec_spec, _vec_spec, _vec_spec],
    out_specs=[_row_spec, _row_spec, _vec_spec],
    out_shape=[
        jax.ShapeDtypeStruct((N, D), jnp.float32),
        jax.ShapeDtypeStruct((N, D), jnp.float32),
        jax.ShapeDtypeStruct((1, D), jnp.float32),
    ],
    scratch_shapes=[pltpu.VMEM((1, D), jnp.float32)],
)


def kernel(x, edge_index, W_in, b_in, layers_W, layers_b, glob_W, glob_b,
           gamma, beta):
    row = edge_index[0].reshape(NW, CH, K)
    col = edge_index[1].reshape(NW, CH, K)
    onesD = jnp.ones((K, D), jnp.float32)
    zeroD = jnp.zeros((RPS, D), jnp.float32)

    _sc_degree, _sc_messages = _get_sc_kernels()
    degp = _sc_degree(row, onesD, zeroD)
    h, hsc, hmean = _tc_input(x, W_in, b_in.reshape(1, D), degp)

    num_layers = layers_W.shape[0]
    for l in range(num_layers):
        msgp = _sc_messages(hsc, col, row, zeroD)
        h, hsc, hmean = _tc_layer(
            h, msgp, degp, hmean,
            glob_W[l], glob_b[l].reshape(1, D),
            layers_W[l], layers_b[l].reshape(1, D),
            gamma[l].reshape(1, D), beta[l].reshape(1, D))
    return h


# speedup vs baseline: 1.3757x; 1.3757x over previous
import functools

import jax
import jax.numpy as jnp
from jax import lax
from jax.experimental import pallas as pl
from jax.experimental.pallas import tpu as pltpu
from jax.experimental.pallas import tpu_sc as plsc

N = 10000
E = 320000
D = 128
NC = 2
NS = 16
NW = NC * NS
EPT = E // NW
K = 100
CH = EPT // K
NBUF = 2
PASSES = 2
CPP = CH // PASSES
NP = 10240
RPS = NP // NS


@functools.cache
def _get_sc_kernels():
    mesh = plsc.VectorSubcoreMesh(core_axis_name="c", subcore_axis_name="s")

    @functools.partial(
        pl.kernel,
        out_type=jax.ShapeDtypeStruct((NC, NP, D), jnp.float32),
        mesh=mesh,
        scratch_types=[
            pltpu.VMEM((CH, K), jnp.int32),
            pltpu.VMEM((K, D), jnp.float32),
            pltpu.VMEM_SHARED((NP, D), jnp.float32),
        ],
    )
    def _sc_degree(row_hbm, ones_hbm, zero_hbm, out_hbm, row_v, ones_v, acc):
        c = lax.axis_index("c")
        s = lax.axis_index("s")
        cid = c * NS + s
        pltpu.sync_copy(zero_hbm, acc.at[pl.ds(s * RPS, RPS)])
        pltpu.sync_copy(ones_hbm, ones_v)
        pltpu.sync_copy(row_hbm.at[cid], row_v)
        plsc.subcore_barrier()

        def body(j, _):
            pltpu.sync_copy(ones_v, acc.at[row_v.at[j]], add=True)
            return ()

        lax.fori_loop(0, CH, body, (), unroll=False)
        plsc.subcore_barrier()
        pltpu.sync_copy(acc.at[pl.ds(s * RPS, RPS)],
                        out_hbm.at[c, pl.ds(s * RPS, RPS)])

    @functools.partial(
        pl.kernel,
        out_type=jax.ShapeDtypeStruct((NC, NP, D), jnp.float32),
        mesh=mesh,
        scratch_types=[
            pltpu.VMEM((CPP, K), jnp.int32),
            pltpu.VMEM((CPP, K), jnp.int32),
            pltpu.VMEM((NBUF, K, D), jnp.float32),
            pltpu.VMEM_SHARED((NP, D), jnp.float32),
            [pltpu.SemaphoreType.DMA] * NBUF,
        ],
    )
    def _sc_messages(hsc_hbm, col0_hbm, col1_hbm, row0_hbm, row1_hbm,
                     zero_hbm, out_hbm, col_v, row_v, rows_v, acc, sems):
        c = lax.axis_index("c")
        s = lax.axis_index("s")
        cid = c * NS + s
        pltpu.sync_copy(zero_hbm, acc.at[pl.ds(s * RPS, RPS)])
        plsc.subcore_barrier()

        for col_hbm, row_hbm in ((col0_hbm, row0_hbm), (col1_hbm, row1_hbm)):
            pltpu.sync_copy(col_hbm.at[cid], col_v)
            pltpu.sync_copy(row_hbm.at[cid], row_v)

            for b in range(NBUF):
                pltpu.async_copy(hsc_hbm.at[col_v.at[b]], rows_v.at[b],
                                 sems[b])

            def body(t, _):
                for b in range(NBUF):
                    j = t * NBUF + b
                    pltpu.make_async_copy(
                        hsc_hbm.at[col_v.at[b]], rows_v.at[b], sems[b]).wait()
                    pltpu.async_copy(
                        rows_v.at[b], acc.at[row_v.at[j]], sems[b], add=True
                    ).wait()
                    jn = (j + NBUF) % CPP
                    pltpu.async_copy(
                        hsc_hbm.at[col_v.at[jn]], rows_v.at[b], sems[b])
                return ()

            lax.fori_loop(0, CPP // NBUF, body, (), unroll=False)
            for b in range(NBUF):
                pltpu.make_async_copy(
                    hsc_hbm.at[col_v.at[b]], rows_v.at[b], sems[b]).wait()
        plsc.subcore_barrier()
        pltpu.sync_copy(acc.at[pl.ds(s * RPS, RPS)],
                        out_hbm.at[c, pl.ds(s * RPS, RPS)])

    return _sc_degree, _sc_messages


BLK = 2000
GRID = N // BLK


def _tc_input_body(x_ref, w_ref, b_ref, dp_ref, h_ref, hsc_ref, mean_ref, acc):
    i = pl.program_id(0)
    h = jnp.dot(x_ref[...], w_ref[...],
                preferred_element_type=jnp.float32) + b_ref[...]
    deg = dp_ref[0, :, 0:1] + dp_ref[1, :, 0:1] + 1.0
    dis = lax.rsqrt(deg)
    h_ref[...] = h
    hsc_ref[...] = h * dis

    @pl.when(i == 0)
    def _():
        acc[...] = jnp.zeros_like(acc)

    acc[...] += jnp.sum(h, axis=0, keepdims=True)

    @pl.when(i == GRID - 1)
    def _():
        mean_ref[...] = acc[...] * (1.0 / N)


def _tc_layer_body(h_ref, mp_ref, dp_ref, mean_ref, gw_ref, gb_ref,
                   w_ref, b_ref, gamma_ref, beta_ref,
                   hn_ref, hsc_ref, mean_out_ref, acc):
    i = pl.program_id(0)
    deg = dp_ref[0, :, 0:1] + dp_ref[1, :, 0:1] + 1.0
    dis = lax.rsqrt(deg)
    msgs = (mp_ref[0] + mp_ref[1]) * dis
    ctx = jnp.dot(mean_ref[...], gw_ref[...],
                  preferred_element_type=jnp.float32) + gb_ref[...]
    z = jnp.dot(h_ref[...] + msgs, w_ref[...],
                preferred_element_type=jnp.float32) + b_ref[...] + ctx
    z = jnp.maximum(z, 0.0)
    mu = jnp.mean(z, axis=-1, keepdims=True)
    var = jnp.mean(jnp.square(z - mu), axis=-1, keepdims=True)
    hn = (z - mu) * lax.rsqrt(var + 1e-5) * gamma_ref[...] + beta_ref[...]
    hn_ref[...] = hn
    hsc_ref[...] = hn * dis

    @pl.when(i == 0)
    def _():
        acc[...] = jnp.zeros_like(acc)

    acc[...] += jnp.sum(hn, axis=0, keepdims=True)

    @pl.when(i == GRID - 1)
    def _():
        mean_out_ref[...] = acc[...] * (1.0 / N)


_row_spec = pl.BlockSpec((BLK, D), lambda i: (i, 0))
_full_spec = pl.BlockSpec((D, D), lambda i: (0, 0))
_vec_spec = pl.BlockSpec((1, D), lambda i: (0, 0))
_dp_spec = pl.BlockSpec((NC, BLK, D), lambda i: (0, i, 0))
_mp_spec = pl.BlockSpec((NC, BLK, D), lambda i: (0, i, 0))

_tc_input = pl.pallas_call(
    _tc_input_body,
    grid=(GRID,),
    in_specs=[_row_spec, _full_spec, _vec_spec, _dp_spec],
    out_specs=[_row_spec, _row_spec, _vec_spec],
    out_shape=[
        jax.ShapeDtypeStruct((N, D), jnp.float32),
        jax.ShapeDtypeStruct((N, D), jnp.float32),
        jax.ShapeDtypeStruct((1, D), jnp.float32),
    ],
    scratch_shapes=[pltpu.VMEM((1, D), jnp.float32)],
)

_tc_layer = pl.pallas_call(
    _tc_layer_body,
    grid=(GRID,),
    in_specs=[_row_spec, _mp_spec, _dp_spec, _vec_spec, _full_spec, _vec_spec,
              _full_spec, _vec_spec, _vec_spec, _vec_spec],
    out_specs=[_row_spec, _row_spec, _vec_spec],
    out_shape=[
        jax.ShapeDtypeStruct((N, D), jnp.float32),
        jax.ShapeDtypeStruct((N, D), jnp.float32),
        jax.ShapeDtypeStruct((1, D), jnp.float32),
    ],
    scratch_shapes=[pltpu.VMEM((1, D), jnp.float32)],
)


def kernel(x, edge_index, W_in, b_in, layers_W, layers_b, glob_W, glob_b,
           gamma, beta):
    row = edge_index[0].reshape(NW, CH, K)
    rowp = edge_index[0].reshape(PASSES, NW, CPP, K)
    colp = edge_index[1].reshape(PASSES, NW, CPP, K)
    row0, row1 = rowp[0], rowp[1]
    col0, col1 = colp[0], colp[1]
    onesD = jnp.ones((K, D), jnp.float32)
    zeroD = jnp.zeros((RPS, D), jnp.float32)

    _sc_degree, _sc_messages = _get_sc_kernels()
    degp = _sc_degree(row, onesD, zeroD)
    h, hsc, hmean = _tc_input(x, W_in, b_in.reshape(1, D), degp)

    num_layers = layers_W.shape[0]
    for l in range(num_layers):
        msgp = _sc_messages(hsc, col0, col1, row0, row1, zeroD)
        h, hsc, hmean = _tc_layer(
            h, msgp, degp, hmean,
            glob_W[l], glob_b[l].reshape(1, D),
            layers_W[l], layers_b[l].reshape(1, D),
            gamma[l].reshape(1, D), beta[l].reshape(1, D))
    return h
